# Initial kernel scaffold; baseline (speedup 1.0000x reference)
import jax
import jax.numpy as jnp
from jax.experimental import pallas as pl


def kernel(feas_sim, scores):
    raise NotImplementedError("write your pallas kernel here")


# speedup vs baseline: 3.8225x; 3.8225x over previous
import functools

import jax
import jax.numpy as jnp
from jax.experimental import pallas as pl
from jax.experimental.pallas import tpu as pltpu

N = 4096
D = 128
C = 100
K = 100
KC = 20
NUM_TRUE_TH = 90
SCORE_TH = 0.99

ROWS = 256


def _topk_body(feas_blk, feas_all, scoresT, scores_blk,
               sk_ref, nt_ref, it_ref, simbuf):
    R = ROWS
    sim = jax.lax.dot_general(
        feas_blk[...], feas_all[...],
        (((1,), (1,)), ((), ())),
        preferred_element_type=jnp.float32,
        precision=jax.lax.Precision.DEFAULT,
    )
    simbuf[...] = sim

    iota_l = jax.lax.broadcasted_iota(jnp.int32, (R, N), 1)
    iota_k = jax.lax.broadcasted_iota(jnp.int32, (R, K), 1)

    st = scoresT[...]
    mt = jnp.max(st, axis=0, keepdims=True)
    iota_c = jax.lax.broadcasted_iota(jnp.int32, (C, N), 0)
    labels_all = jnp.min(jnp.where(st == mt, iota_c, C), axis=0,
                         keepdims=True)

    m0 = jnp.max(sim, axis=1, keepdims=True)
    j0 = jnp.min(jnp.where(sim == m0, iota_l, N), axis=1, keepdims=True)
    top1_label = jnp.sum(jnp.where(iota_l == j0, labels_all, 0), axis=1,
                         keepdims=True)

    sk_ref[...] = jnp.zeros((R, K), jnp.float32)

    def body(r, carry):
        s = simbuf[...]
        m = jnp.max(s, axis=1, keepdims=True)
        j = jnp.min(jnp.where(s == m, iota_l, N), axis=1, keepdims=True)
        onehot = iota_l == j
        simbuf[...] = jnp.where(onehot, -2.0, s)
        sk_ref[...] += jnp.where(iota_k == r, m, 0.0)
        return carry

    jax.lax.fori_loop(0, K, body, 0, unroll=False)

    mask = simbuf[...] < -1.5
    nt = jnp.sum(jnp.where(mask & (labels_all == top1_label), 1, 0),
                 axis=1, keepdims=True)
    conf = jnp.max(scores_blk[...], axis=1, keepdims=True)
    nt_ref[...] = nt
    it_ref[...] = (nt >= NUM_TRUE_TH) & (conf > SCORE_TH)


def _centers_body(scoresT_ref, feas_ref, out_ref, sbuf, wbuf):
    sbuf[...] = scoresT_ref[...]
    wbuf[...] = jnp.zeros((C, N), jnp.float32)
    iota_l = jax.lax.broadcasted_iota(jnp.int32, (C, N), 1)

    def body(r, carry):
        s = sbuf[...]
        m = jnp.max(s, axis=1, keepdims=True)
        j = jnp.min(jnp.where(s == m, iota_l, N), axis=1, keepdims=True)
        onehot = iota_l == j
        sbuf[...] = jnp.where(onehot, -1.0, s)
        wbuf[...] += onehot.astype(jnp.float32)
        return carry

    jax.lax.fori_loop(0, KC, body, 0, unroll=False)

    acc = jax.lax.dot_general(
        wbuf[...], feas_ref[...],
        (((1,), (0,)), ((), ())),
        preferred_element_type=jnp.float32,
        precision=jax.lax.Precision.HIGHEST,
    )
    out_ref[...] = acc / jnp.float32(KC)


@functools.partial(jax.jit, static_argnames=("interpret",))
def kernel(feas_sim, scores, interpret=False):
    scoresT = scores.T

    grid = N // ROWS
    scores_k, num_true, idx_true = pl.pallas_call(
        _topk_body,
        grid=(grid,),
        in_specs=[
            pl.BlockSpec((ROWS, D), lambda i: (i, 0)),
            pl.BlockSpec((N, D), lambda i: (0, 0)),
            pl.BlockSpec((C, N), lambda i: (0, 0)),
            pl.BlockSpec((ROWS, C), lambda i: (i, 0)),
        ],
        out_specs=[
            pl.BlockSpec((ROWS, K), lambda i: (i, 0)),
            pl.BlockSpec((ROWS, 1), lambda i: (i, 0)),
            pl.BlockSpec((ROWS, 1), lambda i: (i, 0)),
        ],
        out_shape=[
            jax.ShapeDtypeStruct((N, K), jnp.float32),
            jax.ShapeDtypeStruct((N, 1), jnp.int32),
            jax.ShapeDtypeStruct((N, 1), jnp.bool_),
        ],
        scratch_shapes=[pltpu.VMEM((ROWS, N), jnp.float32)],
        interpret=interpret,
    )(feas_sim, feas_sim, scoresT, scores)

    centers = pl.pallas_call(
        _centers_body,
        out_shape=jax.ShapeDtypeStruct((C, D), jnp.float32),
        scratch_shapes=[
            pltpu.VMEM((C, N), jnp.float32),
            pltpu.VMEM((C, N), jnp.float32),
        ],
        interpret=interpret,
    )(scoresT, feas_sim)

    return centers, scores_k, num_true[:, 0], idx_true[:, 0]
